# baseline (speedup 1.0000x reference)
import functools

import jax
import jax.numpy as jnp
from jax import lax
from jax.experimental import pallas as pl
from jax.experimental.pallas import tpu as pltpu
from jax.experimental.pallas import tpu_sc as plsc

_NC = 2
_NS = 16
_NW = _NC * _NS
_LANES = 16

_CHUNK = 128
_NBUF = 8


@functools.lru_cache(maxsize=None)
def _make_gather(B, F, V, D):
  n = B * F
  rows_w = n // _NW
  nch = rows_w // _CHUNK
  ngrp = nch // _NBUF
  assert rows_w % (_CHUNK * _NBUF) == 0 and _CHUNK % _LANES == 0

  mesh = plsc.VectorSubcoreMesh(
      core_axis_name="c", subcore_axis_name="s",
      num_cores=_NC, num_subcores=_NS)

  @functools.partial(
      pl.kernel,
      out_type=jax.ShapeDtypeStruct((n, D), jnp.float32),
      mesh=mesh,
      scratch_types=[
          pltpu.VMEM((nch, _CHUNK), jnp.int32),
          *[pltpu.VMEM((_CHUNK, D), jnp.float32) for _ in range(_NBUF)],
          *[pltpu.SemaphoreType.DMA for _ in range(_NBUF)],
      ],
      compiler_params=pltpu.CompilerParams(use_tc_tiling_on_sc=False),
  )
  def gather_kernel(xin_hbm, table_hbm, out_hbm, idx_v, *bufs_sems):
    bufs = bufs_sems[:_NBUF]
    sems = bufs_sems[_NBUF:]
    wid = lax.axis_index("s") * _NC + lax.axis_index("c")
    base = wid * rows_w

    pltpu.sync_copy(xin_hbm.at[wid], idx_v)

    vecs_per_chunk = _CHUNK // _LANES

    @pl.loop(0, nch)
    def _fix(c):
      for j in range(vecs_per_chunk):
        pos = base + c * _CHUNK + j * _LANES + lax.iota(jnp.int32, _LANES)
        f = lax.rem(pos, F)
        sl = pl.ds(j * _LANES, _LANES)
        idx_v[c, sl] = idx_v[c, sl] + f * V

    def start(c, slot):
      pltpu.async_copy(table_hbm.at[idx_v.at[c]], bufs[slot], sems[slot])

    def drain(c, slot):
      pltpu.make_async_copy(
          table_hbm.at[idx_v.at[c]], bufs[slot], sems[slot]).wait()
      pltpu.sync_copy(bufs[slot], out_hbm.at[pl.ds(base + c * _CHUNK, _CHUNK)])

    for b in range(_NBUF):
      start(b, b)

    @pl.loop(0, ngrp - 1)
    def _grp(g):
      c0 = g * _NBUF
      for b in range(_NBUF):
        drain(c0 + b, b)
        start(c0 + _NBUF + b, b)

    c0 = (ngrp - 1) * _NBUF
    for b in range(_NBUF):
      drain(c0 + b, b)

  return gather_kernel


def _stats_body(x_ref, o_ref):
  @pl.when(pl.program_id(0) == 0)
  def _():
    o_ref[...] = jnp.zeros_like(o_ref)

  xb = x_ref[...]
  o_ref[0:1, :] += jnp.sum(xb, axis=0, keepdims=True)
  o_ref[1:2, :] += jnp.sum(xb * xb, axis=0, keepdims=True)


def _head_body(nb_inv, x_ref, st_ref, g_ref, be_ref, w_ref, b_ref, o_ref):
  mean = st_ref[0:1, :] * nb_inv
  var = st_ref[1:2, :] * nb_inv - mean * mean
  scale = g_ref[...] * lax.rsqrt(var + 1e-5)
  shift = be_ref[...] - mean * scale
  xn = x_ref[...] * scale + shift
  y = lax.dot_general(xn, w_ref[...], (((1,), (1,)), ((), ())),
                      preferred_element_type=jnp.float32)
  o_ref[...] = jnp.maximum(y + b_ref[...], 0.0)


@functools.lru_cache(maxsize=None)
def _make_head(B, K, OUT, blk):
  nb = B // blk
  stats = pl.pallas_call(
      _stats_body,
      grid=(nb,),
      in_specs=[pl.BlockSpec((blk, K), lambda i: (i, 0))],
      out_specs=pl.BlockSpec((2, K), lambda i: (0, 0)),
      out_shape=jax.ShapeDtypeStruct((2, K), jnp.float32),
  )
  head = pl.pallas_call(
      functools.partial(_head_body, 1.0 / B),
      grid=(nb,),
      in_specs=[
          pl.BlockSpec((blk, K), lambda i: (i, 0)),
          pl.BlockSpec((2, K), lambda i: (0, 0)),
          pl.BlockSpec((1, K), lambda i: (0, 0)),
          pl.BlockSpec((1, K), lambda i: (0, 0)),
          pl.BlockSpec((OUT, K), lambda i: (0, 0)),
          pl.BlockSpec((1, OUT), lambda i: (0, 0)),
      ],
      out_specs=pl.BlockSpec((blk, OUT), lambda i: (i, 0)),
      out_shape=jax.ShapeDtypeStruct((B, OUT), jnp.float32),
  )
  return stats, head


def kernel(x_in, emb_tables, bn_gamma, bn_beta, W, b):
  B, F = x_in.shape
  _, V, D = emb_tables.shape
  OUT = W.shape[0]
  K = F * D

  n = B * F
  rows_w = n // _NW
  xin3 = x_in.reshape(_NW, rows_w // _CHUNK, _CHUNK)
  table_flat = emb_tables.reshape(F * V, D)

  gathered = _make_gather(B, F, V, D)(xin3, table_flat)
  x2d = gathered.reshape(B, K)

  stats_call, head_call = _make_head(B, K, OUT, 2048)
  st = stats_call(x2d)
  return head_call(x2d, st, bn_gamma.reshape(1, K), bn_beta.reshape(1, K),
                   W, b.reshape(1, OUT))
